# baseline (speedup 1.0000x reference)
import jax
import jax.numpy as jnp
from jax import lax
from jax.experimental import pallas as pl
from jax.experimental.pallas import tpu as pltpu

_FRACS = (0.25, 0.5, 0.75, 0.9)


def _mask_kernel(zo_ref, x_ref, o0, o1, o2, o3, *, row_off, col_off, tt):
    v = x_ref[:, row_off, col_off:col_off + tt]
    vi = v[:, :, None]
    vj = v[:, None, :]
    less = (vj < vi).astype(jnp.int32)
    j_lt_i = lax.broadcasted_iota(jnp.int32, (tt, tt), 1) < \
        lax.broadcasted_iota(jnp.int32, (tt, tt), 0)
    eq = jnp.logical_and(vj == vi, j_lt_i[None]).astype(jnp.int32)
    rank = jnp.sum(less + eq, axis=-1)
    zo = zo_ref[0, 0]
    for out, frac in zip((o0, o1, o2, o3), _FRACS):
        thr = int(tt * frac) + zo
        out[...] = jnp.minimum(
            jnp.maximum(rank - thr + 1, 0), 1).astype(jnp.int8)


def kernel(attn_avg, inference, template_token_num):
    B, S, L = attn_avg.shape
    T = L - 200
    tt = 49
    row = (S - 200) + (14 // 2) * 14 + 14 // 2
    col0 = T - tt
    r_blk = row // 8
    c_blk = col0 // 128
    assert col0 + tt <= (c_blk + 1) * 128
    zero_offset = jnp.reshape(
        jnp.asarray(template_token_num, jnp.int32) - T, (1, 1))

    out_sd = jax.ShapeDtypeStruct((B, tt), jnp.int8)
    outs = pl.pallas_call(
        lambda zo, x, o0, o1, o2, o3: _mask_kernel(
            zo, x, o0, o1, o2, o3,
            row_off=row - r_blk * 8, col_off=col0 - c_blk * 128, tt=tt),
        grid=(1,),
        in_specs=[
            pl.BlockSpec(memory_space=pltpu.SMEM),
            pl.BlockSpec((B, 8, 128), lambda i: (0, r_blk, c_blk)),
        ],
        out_specs=[pl.BlockSpec((B, tt), lambda i: (0, 0))] * 4,
        out_shape=[out_sd] * 4,
    )(zero_offset, attn_avg)
    return tuple(o.astype(jnp.bool_) for o in outs)


# speedup vs baseline: 1.0490x; 1.0490x over previous
import jax
import jax.numpy as jnp
from jax import lax
from jax.experimental import pallas as pl
from jax.experimental.pallas import tpu as pltpu

_FRACS = (0.25, 0.5, 0.75, 0.9)


def _mask_kernel(zo_ref, x_ref, o0, o1, o2, o3, *, row_off, col_off, tt):
    v = x_ref[:, row_off, col_off:col_off + tt]
    vi = v[:, None, :]
    vj = v[:, :, None]
    less = (vj < vi).astype(jnp.int32)
    j_lt_i = lax.broadcasted_iota(jnp.int32, (tt, tt), 0) < \
        lax.broadcasted_iota(jnp.int32, (tt, tt), 1)
    eq = jnp.logical_and(vj == vi, j_lt_i[None]).astype(jnp.int32)
    rank = jnp.sum(less + eq, axis=1)
    zo = zo_ref[0, 0]
    for out, frac in zip((o0, o1, o2, o3), _FRACS):
        out[...] = rank >= int(tt * frac) + zo


def kernel(attn_avg, inference, template_token_num):
    B, S, L = attn_avg.shape
    T = L - 200
    tt = 49
    row = (S - 200) + (14 // 2) * 14 + 14 // 2
    col0 = T - tt
    r_blk = row // 8
    c_blk = col0 // 128
    assert col0 + tt <= (c_blk + 1) * 128
    zero_offset = jnp.reshape(
        jnp.asarray(template_token_num, jnp.int32) - T, (1, 1))

    out_sd = jax.ShapeDtypeStruct((B, tt), jnp.bool_)
    outs = pl.pallas_call(
        lambda zo, x, o0, o1, o2, o3: _mask_kernel(
            zo, x, o0, o1, o2, o3,
            row_off=row - r_blk * 8, col_off=col0 - c_blk * 128, tt=tt),
        grid=(1,),
        in_specs=[
            pl.BlockSpec(memory_space=pltpu.SMEM),
            pl.BlockSpec((B, 8, 128), lambda i: (0, r_blk, c_blk)),
        ],
        out_specs=[pl.BlockSpec((B, tt), lambda i: (0, 0))] * 4,
        out_shape=[out_sd] * 4,
    )(zero_offset, attn_avg)
    return tuple(outs)
